# baseline (speedup 1.0000x reference)
import functools

import jax
import jax.numpy as jnp
from jax import lax
from jax.experimental import pallas as pl
from jax.experimental.pallas import tpu as pltpu
from jax.experimental.pallas import tpu_sc as plsc

N_IN = 500000
N_OUT = 1000000
D = 32
NC = 2
NS = 16
NW = NC * NS
CHUNK = 128
SRCW = 2176
DSTW = CHUNK * D
NFULL = N_OUT // CHUNK
NEXTRA = NFULL % NW
NBASE = NFULL // NW
TAIL = N_OUT - NFULL * CHUNK
TAIL_BASE = NFULL * CHUNK
TAIL_W = 4

_mesh = plsc.VectorSubcoreMesh(core_axis_name="c", subcore_axis_name="s")


def _smap(r):
    return (r + ((r >> 1) & 1)) >> 1


@functools.partial(
    pl.kernel,
    mesh=_mesh,
    out_type=jax.ShapeDtypeStruct((N_OUT * D,), jnp.float32),
    scratch_types=[
        pltpu.VMEM((SRCW,), jnp.float32),
        pltpu.VMEM((SRCW,), jnp.float32),
        pltpu.VMEM((DSTW,), jnp.float32),
        pltpu.VMEM((DSTW,), jnp.float32),
        pltpu.SemaphoreType.DMA,
        pltpu.SemaphoreType.DMA,
        pltpu.SemaphoreType.DMA,
        pltpu.SemaphoreType.DMA,
    ],
)
def _stretch(data_hbm, out_hbm, src0, src1, dst0, dst1, rs0, rs1, ws0, ws1):
    wid = lax.axis_index("s") * NC + lax.axis_index("c")
    count = NBASE + jnp.where(wid < NEXTRA, 1, 0)

    def cidx(i):
        return wid + i * NW

    def fire_read(src, rsem, i):
        pltpu.async_copy(data_hbm.at[pl.ds(cidx(i) * (CHUNK // 2 * D), SRCW)],
                         src, rsem)

    def wait_read(src, rsem):
        pltpu.make_async_copy(data_hbm.at[pl.ds(0, SRCW)], src, rsem).wait()

    def rearrange(src, dst, nrows, cap):
        for r in range(nrows):
            s = min(_smap(r), cap)
            for h in range(0, D, 16):
                dst[pl.ds(r * D + h, 16)] = src[pl.ds(s * D + h, 16)]

    def fire_write(dst, wsem, i):
        pltpu.async_copy(dst, out_hbm.at[pl.ds(cidx(i) * DSTW, DSTW)], wsem)

    def wait_write(dst, wsem):
        pltpu.make_async_copy(dst, out_hbm.at[pl.ds(0, DSTW)], wsem).wait()

    fire_read(src0, rs0, 0)
    fire_read(src1, rs1, 1)

    def step(src, dst, rsem, wsem, i, first):
        wait_read(src, rsem)

        @pl.when(jnp.logical_not(first))
        def _():
            wait_write(dst, wsem)

        rearrange(src, dst, CHUNK, SRCW // D - 1)
        fire_write(dst, wsem, i)

        @pl.when(i + 2 < count)
        def _():
            fire_read(src, rsem, i + 2)

    def body(p, carry):
        i0, i1 = 2 * p, 2 * p + 1

        @pl.when(i0 < count)
        def _():
            step(src0, dst0, rs0, ws0, i0, p == 0)

        @pl.when(i1 < count)
        def _():
            step(src1, dst1, rs1, ws1, i1, p == 0)

        return carry

    lax.fori_loop(0, (NBASE + 2) // 2, body, 0)

    wait_write(dst0, ws0)
    wait_write(dst1, ws1)

    @pl.when(wid == TAIL_W)
    def _():
        pltpu.async_copy(
            data_hbm.at[pl.ds(TAIL_BASE // 2 * D, TAIL // 2 * D)],
            src0.at[pl.ds(0, TAIL // 2 * D)], rs0).wait()
        rearrange(src0, dst0, TAIL, TAIL // 2 - 1)
        pltpu.sync_copy(dst0.at[pl.ds(0, TAIL * D)],
                        out_hbm.at[pl.ds(TAIL_BASE * D, TAIL * D)])


def kernel(data):
    flat = _stretch(data.reshape(N_IN * D))
    return flat.reshape(N_OUT, D)
